# baseline (speedup 1.0000x reference)
import jax
import jax.numpy as jnp
from jax import lax
from jax.experimental import pallas as pl
from jax.experimental.pallas import tpu as pltpu
from jax.experimental.pallas import tpu_sc as plsc

MAX_LEN = 2048
D_K = 32
NSHIFT = 8

_info = plsc.get_sparse_core_info()
_NC, _NS = _info.num_cores, _info.num_subcores
_NW = _NC * _NS
ROWS_PER_W = MAX_LEN // _NW


def _sc_body(tables8_hbm, out_hbm, sem):
    wid = lax.axis_index("s") * _NC + lax.axis_index("c")
    base = wid * ROWS_PER_W
    copies = []
    for r in range(ROWS_PER_W):
        i = base + r
        start = (MAX_LEN - 1) - i
        delta = lax.rem(start, NSHIFT)
        aligned = pl.multiple_of(start - delta, NSHIFT)
        copies.append(
            pltpu.async_copy(
                tables8_hbm.at[delta, pl.ds(aligned, MAX_LEN)],
                out_hbm.at[i],
                sem,
            )
        )
    for c in copies:
        c.wait()


@jax.jit
def _run(rel_embeddings):
    padded = jnp.pad(rel_embeddings, ((0, NSHIFT + 1), (0, 0)))
    tables8 = jnp.stack(
        [lax.slice_in_dim(padded, d, d + 2 * MAX_LEN, axis=0)
         for d in range(NSHIFT)]
    )
    k = pl.kernel(
        _sc_body,
        out_type=jax.ShapeDtypeStruct((MAX_LEN, MAX_LEN, D_K), jnp.float32),
        mesh=plsc.VectorSubcoreMesh(core_axis_name="c", subcore_axis_name="s"),
        scratch_types=[
            pltpu.SemaphoreType.DMA,
        ],
    )
    return k(tables8)


def kernel(seq_len, rel_embeddings):
    del seq_len
    return _run(rel_embeddings)


# speedup vs baseline: 83.0037x; 83.0037x over previous
import jax
import jax.numpy as jnp
from jax import lax
from jax.experimental import pallas as pl
from jax.experimental.pallas import tpu as pltpu
from jax.experimental.pallas import tpu_sc as plsc

MAX_LEN = 2048
D_K = 32
NSHIFT = 128

_info = plsc.get_sparse_core_info()
_NC, _NS = _info.num_cores, _info.num_subcores
_NW = _NC * _NS
ROWS_PER_W = MAX_LEN // _NW
CW = 1024
NCHUNK = MAX_LEN // CW


def _sc_body(tablesT_hbm, out_hbm, buf_v, isem, osem):
    wid = lax.axis_index("s") * _NC + lax.axis_index("c")
    base = wid * ROWS_PER_W
    n_iters = ROWS_PER_W * NCHUNK

    @pl.loop(0, n_iters, step=2)
    def _(it):
        for b in range(2):
            t = it + b

            @pl.when(t >= 2)
            def _():
                pltpu.make_async_copy(
                    buf_v.at[b], out_hbm.at[base, :, pl.ds(0, CW)], osem
                ).wait()

            r = t // NCHUNK
            c = t - r * NCHUNK
            i = base + r
            start = (MAX_LEN - 1) - i
            delta = lax.rem(start, NSHIFT)
            aligned = pl.multiple_of(start - delta + c * CW, NSHIFT)
            pltpu.async_copy(
                tablesT_hbm.at[delta, :, pl.ds(aligned, CW)],
                buf_v.at[b],
                isem,
            ).wait()
            pltpu.async_copy(
                buf_v.at[b],
                out_hbm.at[i, :, pl.ds(c * CW, CW)],
                osem,
            )

    for b in range(2):
        pltpu.make_async_copy(
            buf_v.at[b], out_hbm.at[base, :, pl.ds(0, CW)], osem
        ).wait()


@jax.jit
def _run(rel_embeddings):
    tableT = rel_embeddings.T
    big = jnp.pad(tableT, ((0, 0), (0, NSHIFT + 2)))
    tablesT = jax.vmap(
        lambda d: lax.dynamic_slice(big, (0, d), (D_K, 2 * MAX_LEN))
    )(jnp.arange(NSHIFT))
    k = pl.kernel(
        _sc_body,
        out_type=jax.ShapeDtypeStruct((MAX_LEN, D_K, MAX_LEN), jnp.float32),
        mesh=plsc.VectorSubcoreMesh(core_axis_name="c", subcore_axis_name="s"),
        scratch_types=[
            pltpu.VMEM((2, D_K, CW), jnp.float32),
            pltpu.SemaphoreType.DMA,
            pltpu.SemaphoreType.DMA,
        ],
    )
    out_phys = k(tablesT)
    return jnp.transpose(out_phys, (0, 2, 1))


def kernel(seq_len, rel_embeddings):
    del seq_len
    return _run(rel_embeddings)


# speedup vs baseline: 113.1156x; 1.3628x over previous
import jax
import jax.numpy as jnp
from jax import lax
from jax.experimental import pallas as pl
from jax.experimental.pallas import tpu as pltpu
from jax.experimental.pallas import tpu_sc as plsc

MAX_LEN = 2048
D_K = 32
NSHIFT = 128

_info = plsc.get_sparse_core_info()
_NC, _NS = _info.num_cores, _info.num_subcores
_NW = _NC * _NS
RES_PER_W = NSHIFT // _NW
ROWS_PER_RES = MAX_LEN // NSHIFT
WIN_W = (ROWS_PER_RES - 1) * NSHIFT + MAX_LEN


def _sc_body(tablesT_hbm, out_hbm, win_v, isem, osem):
    wid = lax.axis_index("s") * _NC + lax.axis_index("c")
    for dd in range(RES_PER_W):
        delta = wid * RES_PER_W + dd
        pltpu.async_copy(
            tablesT_hbm.at[delta, :, pl.ds(0, WIN_W)], win_v, isem
        ).wait()
        copies = []
        for m in range(ROWS_PER_RES):
            copies.append(
                pltpu.async_copy(
                    win_v.at[:, pl.ds((ROWS_PER_RES - 1 - m) * NSHIFT, MAX_LEN)],
                    out_hbm.at[(NSHIFT - 1) - delta + NSHIFT * m],
                    osem,
                )
            )
        for c in copies:
            c.wait()


@jax.jit
def _run(rel_embeddings):
    tableT = rel_embeddings.T
    big = jnp.pad(tableT, ((0, 0), (0, NSHIFT + 2)))
    tablesT = jax.vmap(
        lambda d: lax.dynamic_slice(big, (0, d), (D_K, 2 * MAX_LEN))
    )(jnp.arange(NSHIFT))
    k = pl.kernel(
        _sc_body,
        out_type=jax.ShapeDtypeStruct((MAX_LEN, D_K, MAX_LEN), jnp.float32),
        mesh=plsc.VectorSubcoreMesh(core_axis_name="c", subcore_axis_name="s"),
        scratch_types=[
            pltpu.VMEM((D_K, WIN_W), jnp.float32),
            pltpu.SemaphoreType.DMA,
            pltpu.SemaphoreType.DMA,
        ],
    )
    out_phys = k(tablesT)
    return jnp.transpose(out_phys, (0, 2, 1))


def kernel(seq_len, rel_embeddings):
    del seq_len
    return _run(rel_embeddings)


# speedup vs baseline: 234.1139x; 2.0697x over previous
import jax
import jax.numpy as jnp
from jax import lax
from jax.experimental import pallas as pl
from jax.experimental.pallas import tpu as pltpu
from jax.experimental.pallas import tpu_sc as plsc

MAX_LEN = 2048
D_K = 32
NSHIFT = 128
TBW = 2 * MAX_LEN
PADW = TBW + NSHIFT

_info = plsc.get_sparse_core_info()
_NC, _NS = _info.num_cores, _info.num_subcores
_NW = _NC * _NS
RES_PER_W = NSHIFT // _NW
ROWS_PER_RES = MAX_LEN // NSHIFT
WIN_W = (ROWS_PER_RES - 1) * NSHIFT + MAX_LEN


def _tc_build_body(big_ref, out_ref):
    d = pl.program_id(0)
    rolled = pltpu.roll(big_ref[:, :], lax.rem(PADW - d, PADW), 1)
    out_ref[0] = rolled[:, :TBW]


def _sc_body(tablesT_hbm, out_hbm, win_v, isem, osem):
    wid = lax.axis_index("s") * _NC + lax.axis_index("c")
    for dd in range(RES_PER_W):
        delta = wid * RES_PER_W + dd
        pltpu.async_copy(
            tablesT_hbm.at[delta, :, pl.ds(0, WIN_W)], win_v, isem
        ).wait()
        copies = []
        for m in range(ROWS_PER_RES):
            copies.append(
                pltpu.async_copy(
                    win_v.at[:, pl.ds((ROWS_PER_RES - 1 - m) * NSHIFT, MAX_LEN)],
                    out_hbm.at[(NSHIFT - 1) - delta + NSHIFT * m],
                    osem,
                )
            )
        for c in copies:
            c.wait()


@jax.jit
def _run(rel_embeddings):
    tableT = rel_embeddings.T
    bigpad = jnp.pad(tableT, ((0, 0), (0, PADW - (2 * MAX_LEN - 1))))
    tablesT = pl.pallas_call(
        _tc_build_body,
        grid=(NSHIFT,),
        in_specs=[pl.BlockSpec((D_K, PADW), lambda d: (0, 0))],
        out_specs=pl.BlockSpec((1, D_K, TBW), lambda d: (d, 0, 0)),
        out_shape=jax.ShapeDtypeStruct((NSHIFT, D_K, TBW), jnp.float32),
    )(bigpad)
    k = pl.kernel(
        _sc_body,
        out_type=jax.ShapeDtypeStruct((MAX_LEN, D_K, MAX_LEN), jnp.float32),
        mesh=plsc.VectorSubcoreMesh(core_axis_name="c", subcore_axis_name="s"),
        scratch_types=[
            pltpu.VMEM((D_K, WIN_W), jnp.float32),
            pltpu.SemaphoreType.DMA,
            pltpu.SemaphoreType.DMA,
        ],
    )
    out_phys = k(tablesT)
    return jnp.transpose(out_phys, (0, 2, 1))


def kernel(seq_len, rel_embeddings):
    del seq_len
    return _run(rel_embeddings)
